# Initial kernel scaffold; baseline (speedup 1.0000x reference)
import jax
import jax.numpy as jnp
from jax.experimental import pallas as pl


def kernel(track_emb, edge_index, sampled_tracks):
    raise NotImplementedError("write your pallas kernel here")


# speedup vs baseline: 1.2394x; 1.2394x over previous
import functools

import jax
import jax.numpy as jnp
from jax import lax
from jax.experimental import pallas as pl
from jax.experimental.pallas import tpu as pltpu
from jax.experimental.pallas import tpu_sc as plsc

P = 10000
E = 320000
D = 128
NC, NS, L = 2, 16, 16
NW = NC * NS

P_PAD = 10240
ROWS_W = P_PAD // NW
RSUB = 80
EW = E // NW
EC = 80
NCHUNK = EW // EC


def _mesh():
    return plsc.VectorSubcoreMesh(core_axis_name="c", subcore_axis_name="s")


def _wid():
    return lax.axis_index("s") * NC + lax.axis_index("c")


@functools.partial(
    pl.kernel,
    mesh=_mesh(),
    out_type=jax.ShapeDtypeStruct((P_PAD, D), jnp.float32),
    scratch_types=[
        pltpu.VMEM((ROWS_W,), jnp.int32),
        pltpu.VMEM((ROWS_W,), jnp.int32),
        pltpu.VMEM((RSUB, D), jnp.float32),
        pltpu.VMEM((RSUB, D), jnp.float32),
        pltpu.SemaphoreType.DMA,
        pltpu.SemaphoreType.DMA,
    ],
)
def _hplay_kernel(emb, s0, s1, hp, i0_v, i1_v, a_v, b_v, sem_a, sem_b):
    wid = _wid()
    base = wid * ROWS_W
    pltpu.sync_copy(s0.at[pl.ds(base, ROWS_W)], i0_v)
    pltpu.sync_copy(s1.at[pl.ds(base, ROWS_W)], i1_v)
    for sub in range(ROWS_W // RSUB):
        ca = pltpu.async_copy(emb.at[i0_v.at[pl.ds(sub * RSUB, RSUB)]], a_v, sem_a)
        cb = pltpu.async_copy(emb.at[i1_v.at[pl.ds(sub * RSUB, RSUB)]], b_v, sem_b)
        ca.wait()
        cb.wait()

        def row_mean(r, _):
            for d8 in range(D // L):
                sl = pl.ds(d8 * L, L)
                a_v[r, sl] = (a_v[r, sl] + b_v[r, sl]) * 0.5
            return 0

        lax.fori_loop(0, RSUB, row_mean, 0)
        pltpu.sync_copy(a_v, hp.at[pl.ds(base + sub * RSUB, RSUB)])


@functools.partial(
    pl.kernel,
    mesh=_mesh(),
    compiler_params=pltpu.CompilerParams(needs_layout_passes=False),
    out_type=jax.ShapeDtypeStruct((E,), jnp.float32),
    scratch_types=[
        pltpu.VMEM((EW,), jnp.int32),
        pltpu.VMEM((EW,), jnp.int32),
        pltpu.VMEM((EW,), jnp.float32),
        pltpu.VMEM((EC, D), jnp.float32),
        pltpu.VMEM((EC, D), jnp.float32),
        pltpu.SemaphoreType.DMA,
        pltpu.SemaphoreType.DMA,
    ],
)
def _score_kernel(hp, emb, src, dst, out, src_v, dst_v, sc_v, a_v, b_v,
                  sem_a, sem_b):
    wid = _wid()
    eb = wid * EW
    pltpu.sync_copy(src.at[pl.ds(eb, EW)], src_v)
    pltpu.sync_copy(dst.at[pl.ds(eb, EW)], dst_v)

    def chunk(c, _):
        off = pl.multiple_of(c * EC, 8)
        ca = pltpu.async_copy(hp.at[src_v.at[pl.ds(off, EC)]], a_v, sem_a)
        cb = pltpu.async_copy(emb.at[dst_v.at[pl.ds(off, EC)]], b_v, sem_b)
        ca.wait()
        cb.wait()
        for g in range(EC // L):
            rows = lax.iota(jnp.int32, L) + g * L
            acc = jnp.zeros((L,), jnp.float32)

            def dstep(d, acc):
                for u in range(8):
                    cols = jnp.full((L,), d * 8 + u, jnp.int32)
                    acc = acc + (plsc.load_gather(a_v, [rows, cols]) *
                                 plsc.load_gather(b_v, [rows, cols]))
                return acc

            acc = lax.fori_loop(0, D // 8, dstep, acc)
            sc_v[pl.ds(off + g * L, L)] = acc
        return 0

    lax.fori_loop(0, NCHUNK, chunk, 0)
    pltpu.sync_copy(sc_v, out.at[pl.ds(eb, EW)])


def kernel(track_emb, edge_index, sampled_tracks):
    track_emb = track_emb.astype(jnp.float32)
    src = edge_index[0].astype(jnp.int32)
    dst = edge_index[1].astype(jnp.int32)
    st = sampled_tracks.astype(jnp.int32)
    s0 = jnp.pad(st[:, 0], (0, P_PAD - P))
    s1 = jnp.pad(st[:, 1], (0, P_PAD - P))
    hp = _hplay_kernel(track_emb, s0, s1)
    return _score_kernel(hp, track_emb, src, dst)


# speedup vs baseline: 1.3932x; 1.1241x over previous
import functools

import jax
import jax.numpy as jnp
from jax import lax
from jax.experimental import pallas as pl
from jax.experimental.pallas import tpu as pltpu
from jax.experimental.pallas import tpu_sc as plsc

P = 10000
E = 320000
D = 128
NC, NS, L = 2, 16, 16
NW = NC * NS

P_PAD = 10240
ROWS_W = P_PAD // NW
RSUB = 80
EW = E // NW
EC = 80
NCHUNK = EW // EC


def _mesh():
    return plsc.VectorSubcoreMesh(core_axis_name="c", subcore_axis_name="s")


def _wid():
    return lax.axis_index("s") * NC + lax.axis_index("c")


@functools.partial(
    pl.kernel,
    mesh=_mesh(),
    out_type=jax.ShapeDtypeStruct((P_PAD, D), jnp.float32),
    scratch_types=[
        pltpu.VMEM((ROWS_W,), jnp.int32),
        pltpu.VMEM((ROWS_W,), jnp.int32),
        pltpu.VMEM((RSUB, D), jnp.float32),
        pltpu.VMEM((RSUB, D), jnp.float32),
        pltpu.SemaphoreType.DMA,
        pltpu.SemaphoreType.DMA,
    ],
)
def _hplay_kernel(emb, s0, s1, hp, i0_v, i1_v, a_v, b_v, sem_a, sem_b):
    wid = _wid()
    base = wid * ROWS_W
    pltpu.sync_copy(s0.at[pl.ds(base, ROWS_W)], i0_v)
    pltpu.sync_copy(s1.at[pl.ds(base, ROWS_W)], i1_v)
    for sub in range(ROWS_W // RSUB):
        ca = pltpu.async_copy(emb.at[i0_v.at[pl.ds(sub * RSUB, RSUB)]], a_v, sem_a)
        cb = pltpu.async_copy(emb.at[i1_v.at[pl.ds(sub * RSUB, RSUB)]], b_v, sem_b)
        ca.wait()
        cb.wait()

        def row_mean(r, _):
            for d8 in range(D // L):
                sl = pl.ds(d8 * L, L)
                a_v[r, sl] = (a_v[r, sl] + b_v[r, sl]) * 0.5
            return 0

        lax.fori_loop(0, RSUB, row_mean, 0)
        pltpu.sync_copy(a_v, hp.at[pl.ds(base + sub * RSUB, RSUB)])


@functools.partial(
    pl.kernel,
    mesh=_mesh(),
    compiler_params=pltpu.CompilerParams(needs_layout_passes=False),
    out_type=jax.ShapeDtypeStruct((E,), jnp.float32),
    scratch_types=[
        pltpu.VMEM((EW,), jnp.int32),
        pltpu.VMEM((EW,), jnp.int32),
        pltpu.VMEM((EW,), jnp.float32),
        *([pltpu.VMEM((EC, D), jnp.float32)] * 8),
        *([pltpu.SemaphoreType.DMA] * 8),
    ],
)
def _score_kernel(hp, emb, src, dst, out, src_v, dst_v, sc_v,
                  a0, a1, a2, a3, b0, b1, b2, b3,
                  sa0, sa1, sa2, sa3, sb0, sb1, sb2, sb3):
    wid = _wid()
    eb = wid * EW
    pltpu.sync_copy(src.at[pl.ds(eb, EW)], src_v)
    pltpu.sync_copy(dst.at[pl.ds(eb, EW)], dst_v)

    a_bufs, b_bufs = (a0, a1, a2, a3), (b0, b1, b2, b3)
    a_sems, b_sems = (sa0, sa1, sa2, sa3), (sb0, sb1, sb2, sb3)
    NBUF = 4

    def idx_a(c):
        return src_v.at[pl.ds(pl.multiple_of(c * EC, 8), EC)]

    def idx_b(c):
        return dst_v.at[pl.ds(pl.multiple_of(c * EC, 8), EC)]

    def issue(c, u):
        pltpu.async_copy(hp.at[idx_a(c)], a_bufs[u], a_sems[u])
        pltpu.async_copy(emb.at[idx_b(c)], b_bufs[u], b_sems[u])

    def wait(c, u):
        pltpu.make_async_copy(hp.at[idx_a(c)], a_bufs[u], a_sems[u]).wait()
        pltpu.make_async_copy(emb.at[idx_b(c)], b_bufs[u], b_sems[u]).wait()

    def compute(c, a_v, b_v):
        off = pl.multiple_of(c * EC, 8)
        for g in range(EC // L):
            rows = lax.iota(jnp.int32, L) + g * L
            acc = jnp.zeros((L,), jnp.float32)

            def dstep(d, acc):
                for uu in range(8):
                    cols = jnp.full((L,), d * 8 + uu, jnp.int32)
                    acc = acc + (plsc.load_gather(a_v, [rows, cols]) *
                                 plsc.load_gather(b_v, [rows, cols]))
                return acc

            acc = lax.fori_loop(0, D // 8, dstep, acc)
            sc_v[pl.ds(off + g * L, L)] = acc

    for j in range(NBUF):
        issue(j, j)

    def ring(i2, _):
        for u in range(NBUF):
            c = i2 * NBUF + u
            wait(c, u)
            compute(c, a_bufs[u], b_bufs[u])

            @pl.when(c + NBUF < NCHUNK)
            def _():
                issue(c + NBUF, u)
        return 0

    lax.fori_loop(0, (NCHUNK - 1) // NBUF, ring, 0)
    last = NCHUNK - 1
    wait(last, last % NBUF)
    compute(last, a_bufs[last % NBUF], b_bufs[last % NBUF])
    pltpu.sync_copy(sc_v, out.at[pl.ds(eb, EW)])


def kernel(track_emb, edge_index, sampled_tracks):
    track_emb = track_emb.astype(jnp.float32)
    src = edge_index[0].astype(jnp.int32)
    dst = edge_index[1].astype(jnp.int32)
    st = sampled_tracks.astype(jnp.int32)
    s0 = jnp.pad(st[:, 0], (0, P_PAD - P))
    s1 = jnp.pad(st[:, 1], (0, P_PAD - P))
    hp = _hplay_kernel(track_emb, s0, s1)
    return _score_kernel(hp, track_emb, src, dst)


# speedup vs baseline: 9.9338x; 7.1301x over previous
import functools

import jax
import jax.numpy as jnp
from jax import lax
from jax.experimental import pallas as pl
from jax.experimental.pallas import tpu as pltpu
from jax.experimental.pallas import tpu_sc as plsc

P = 10000
E = 320000
D = 128
NC, NS, L = 2, 16, 16
NW = NC * NS

P_PAD = 10240
ROWS_W = P_PAD // NW
RSUB = 80
EW = E // NW
EC = 80
NCHUNK = EW // EC


def _mesh():
    return plsc.VectorSubcoreMesh(core_axis_name="c", subcore_axis_name="s")


def _wid():
    return lax.axis_index("s") * NC + lax.axis_index("c")


@functools.partial(
    pl.kernel,
    mesh=_mesh(),
    out_type=jax.ShapeDtypeStruct((P_PAD, D), jnp.float32),
    scratch_types=[
        pltpu.VMEM((ROWS_W,), jnp.int32),
        pltpu.VMEM((ROWS_W,), jnp.int32),
        pltpu.VMEM((RSUB, D), jnp.float32),
        pltpu.VMEM((RSUB, D), jnp.float32),
        pltpu.SemaphoreType.DMA,
        pltpu.SemaphoreType.DMA,
    ],
)
def _hplay_kernel(emb, s0, s1, hp, i0_v, i1_v, a_v, b_v, sem_a, sem_b):
    wid = _wid()
    base = wid * ROWS_W
    pltpu.sync_copy(s0.at[pl.ds(base, ROWS_W)], i0_v)
    pltpu.sync_copy(s1.at[pl.ds(base, ROWS_W)], i1_v)
    for sub in range(ROWS_W // RSUB):
        ca = pltpu.async_copy(emb.at[i0_v.at[pl.ds(sub * RSUB, RSUB)]], a_v, sem_a)
        cb = pltpu.async_copy(emb.at[i1_v.at[pl.ds(sub * RSUB, RSUB)]], b_v, sem_b)
        ca.wait()
        cb.wait()

        def row_mean(r, _):
            for d8 in range(D // L):
                sl = pl.ds(d8 * L, L)
                a_v[r, sl] = (a_v[r, sl] + b_v[r, sl]) * 0.5
            return 0

        lax.fori_loop(0, RSUB, row_mean, 0)
        pltpu.sync_copy(a_v, hp.at[pl.ds(base + sub * RSUB, RSUB)])


@functools.partial(
    pl.kernel,
    mesh=_mesh(),
    compiler_params=pltpu.CompilerParams(needs_layout_passes=False),
    out_type=jax.ShapeDtypeStruct((E,), jnp.float32),
    scratch_types=[
        pltpu.VMEM((EW,), jnp.int32),
        pltpu.VMEM((EW,), jnp.int32),
        pltpu.VMEM((EW,), jnp.float32),
        *([pltpu.VMEM((EC, D), jnp.float32)] * 8),
        *([pltpu.SemaphoreType.DMA] * 8),
    ],
)
def _score_kernel(hp, emb, src, dst, out, src_v, dst_v, sc_v,
                  a0, a1, a2, a3, b0, b1, b2, b3,
                  sa0, sa1, sa2, sa3, sb0, sb1, sb2, sb3):
    wid = _wid()
    eb = wid * EW
    pltpu.sync_copy(src.at[pl.ds(eb, EW)], src_v)
    pltpu.sync_copy(dst.at[pl.ds(eb, EW)], dst_v)

    a_bufs, b_bufs = (a0, a1, a2, a3), (b0, b1, b2, b3)
    a_sems, b_sems = (sa0, sa1, sa2, sa3), (sb0, sb1, sb2, sb3)
    NBUF = 4

    def idx_a(c):
        return src_v.at[pl.ds(pl.multiple_of(c * EC, 8), EC)]

    def idx_b(c):
        return dst_v.at[pl.ds(pl.multiple_of(c * EC, 8), EC)]

    def issue(c, u):
        pltpu.async_copy(hp.at[idx_a(c)], a_bufs[u], a_sems[u])
        pltpu.async_copy(emb.at[idx_b(c)], b_bufs[u], b_sems[u])

    def wait(c, u):
        pltpu.make_async_copy(hp.at[idx_a(c)], a_bufs[u], a_sems[u]).wait()
        pltpu.make_async_copy(emb.at[idx_b(c)], b_bufs[u], b_sems[u]).wait()

    def compute(c, a_v, b_v):
        off = pl.multiple_of(c * EC, 8)
        lane = lax.iota(jnp.int32, L)
        for g in range(EC // L):
            rows = lane + g * L
            acc = jnp.zeros((L,), jnp.float32)

            def dstep(d, acc):
                for uu in range(8):
                    cols = (lane + (d * 8 + uu)) & (D - 1)
                    acc = acc + (plsc.load_gather(a_v, [rows, cols]) *
                                 plsc.load_gather(b_v, [rows, cols]))
                return acc

            acc = lax.fori_loop(0, D // 8, dstep, acc)
            sc_v[pl.ds(off + g * L, L)] = acc

    for j in range(NBUF):
        issue(j, j)

    def ring(i2, _):
        for u in range(NBUF):
            c = i2 * NBUF + u
            wait(c, u)
            compute(c, a_bufs[u], b_bufs[u])

            @pl.when(c + NBUF < NCHUNK)
            def _():
                issue(c + NBUF, u)
        return 0

    lax.fori_loop(0, (NCHUNK - 1) // NBUF, ring, 0)
    last = NCHUNK - 1
    wait(last, last % NBUF)
    compute(last, a_bufs[last % NBUF], b_bufs[last % NBUF])
    pltpu.sync_copy(sc_v, out.at[pl.ds(eb, EW)])


def kernel(track_emb, edge_index, sampled_tracks):
    track_emb = track_emb.astype(jnp.float32)
    src = edge_index[0].astype(jnp.int32)
    dst = edge_index[1].astype(jnp.int32)
    st = sampled_tracks.astype(jnp.int32)
    s0 = jnp.pad(st[:, 0], (0, P_PAD - P))
    s1 = jnp.pad(st[:, 1], (0, P_PAD - P))
    hp = _hplay_kernel(track_emb, s0, s1)
    return _score_kernel(hp, track_emb, src, dst)
